# Initial kernel scaffold; baseline (speedup 1.0000x reference)
import jax
import jax.numpy as jnp
from jax.experimental import pallas as pl


def kernel(x_H, x_C, x_Others, edge_index_H_H, edge_index_H_C, edge_index_H_Others, edge_index_C_H, edge_index_C_C, edge_index_C_Others, edge_index_Others_H, edge_index_Others_C, edge_index_Others_Others, params):
    raise NotImplementedError("write your pallas kernel here")


# speedup vs baseline: 4.3445x; 4.3445x over previous
import functools

import jax
import jax.numpy as jnp
from jax import lax
from jax.experimental import pallas as pl
from jax.experimental.pallas import tpu as pltpu
from jax.experimental.pallas import tpu_sc as plsc

_TYPES = ("H", "C", "Others")
_F = 16
_NC = 2
_NS = 16
_NW = _NC * _NS
_CHUNK = 128
_EPB = _CHUNK * _NW
_ZROWS = 512


def _acc_rows(n):
    return ((n + 8 + 127) // 128) * 128


def _make_sc_scatter(rels, dsts, n_dst):
    mesh = plsc.VectorSubcoreMesh(core_axis_name="c", subcore_axis_name="s")
    out_type = [jax.ShapeDtypeStruct((2, _acc_rows(n_dst[d]), _F),
                                     jnp.float32)
                for d in dsts]
    scratch = ([pltpu.VMEM((_ZROWS, _F), jnp.float32),
                pltpu.VMEM((_CHUNK,), jnp.int32),
                pltpu.VMEM((_CHUNK,), jnp.int32),
                pltpu.VMEM((_CHUNK, _F), jnp.float32)]
               + [pltpu.VMEM_SHARED((_acc_rows(n_dst[d]), _F), jnp.float32)
                  for d in dsts]
               + [pltpu.SemaphoreType.DMA])
    R = len(rels)
    D = len(dsts)

    def body(*refs):
        g = refs[0:R]
        src = refs[R:2 * R]
        dst = refs[2 * R:3 * R]
        outs = refs[3 * R:3 * R + D]
        zbuf, sidx, didx, rows = refs[3 * R + D:3 * R + D + 4]
        accs = refs[3 * R + D + 4:3 * R + D + 4 + D]
        sem = refs[-1]

        cid = lax.axis_index("c")
        sid = lax.axis_index("s")
        wid = sid * _NC + cid

        def _zb(i, c):
            zbuf[i, :] = jnp.zeros((_F,), jnp.float32)
            return c
        lax.fori_loop(0, _ZROWS, _zb, 0)
        for a_i, d in enumerate(dsts):
            per = _acc_rows(n_dst[d]) // _NS
            for off in range(0, per, _ZROWS):
                c = min(_ZROWS, per - off)
                pltpu.sync_copy(zbuf.at[pl.ds(0, c)],
                                accs[a_i].at[pl.ds(sid * per + off, c)])
        plsc.subcore_barrier()

        acc_of = {d: accs[i] for i, d in enumerate(dsts)}
        for r_i, (s_t, d_t, n_src, nck) in enumerate(rels):
            m = nck * _CHUNK
            a = acc_of[d_t]

            def _cb(j, c, r_i=r_i, m=m, a=a):
                base = wid * m + j * _CHUNK
                pltpu.sync_copy(src[r_i].at[pl.ds(base, _CHUNK)], sidx)
                pltpu.async_copy(g[r_i].at[sidx], rows, sem).wait()
                pltpu.sync_copy(dst[r_i].at[pl.ds(base, _CHUNK)], didx)
                pltpu.sync_copy(rows, a.at[didx], add=True)
                return c
            lax.fori_loop(0, nck, _cb, 0)
        plsc.subcore_barrier()

        for a_i, d in enumerate(dsts):
            per = _acc_rows(n_dst[d]) // _NS
            for off in range(0, per, _ZROWS):
                c = min(_ZROWS, per - off)
                r0 = sid * per + off
                pltpu.sync_copy(accs[a_i].at[pl.ds(r0, c)],
                                zbuf.at[pl.ds(0, c)])
                pltpu.sync_copy(zbuf.at[pl.ds(0, c)],
                                outs[a_i].at[cid, pl.ds(r0, c)])

    return pl.kernel(body, out_type=out_type, mesh=mesh,
                     scratch_types=scratch,
                     compiler_params=pltpu.CompilerParams(
                         use_tc_tiling_on_sc=False))


_TCB = 2000


def _enc_call(x, W1, b1, W2, b2, Wrels):
    N = x.shape[0]
    k = len(Wrels)

    def body(x_ref, w1_ref, b1_ref, w2_ref, b2_ref, *rest):
        wr = rest[:k]
        h_ref = rest[k]
        gs = rest[k + 1:]
        z = jnp.maximum(
            jnp.dot(x_ref[...], w1_ref[...],
                    preferred_element_type=jnp.float32) + b1_ref[...], 0.0)
        h = jnp.maximum(
            jnp.dot(z, w2_ref[...],
                    preferred_element_type=jnp.float32) + b2_ref[...], 0.0)
        h_ref[...] = h
        for i in range(k):
            gs[i][...] = jnp.dot(h, wr[i][...],
                                 preferred_element_type=jnp.float32)

    in_specs = ([pl.BlockSpec((_TCB, 128), lambda i: (i, 0)),
                 pl.BlockSpec((128, 32), lambda i: (0, 0)),
                 pl.BlockSpec((1, 32), lambda i: (0, 0)),
                 pl.BlockSpec((32, _F), lambda i: (0, 0)),
                 pl.BlockSpec((1, _F), lambda i: (0, 0))]
                + [pl.BlockSpec((_F, _F), lambda i: (0, 0))] * k)
    out_specs = [pl.BlockSpec((_TCB, _F), lambda i: (i, 0))] * (1 + k)
    out_shape = [jax.ShapeDtypeStruct((N, _F), jnp.float32)] * (1 + k)
    return pl.pallas_call(body, grid=(N // _TCB,), in_specs=in_specs,
                          out_specs=out_specs, out_shape=out_shape)(
        x, W1, b1, W2, b2, *Wrels)


def _comb_call(P, h, Wroots, brels, Wnext):
    N = h.shape[0]
    nb = N // _TCB
    k = len(Wnext)

    def body(p0_ref, p1_ref, h_ref, wr0, wr1, wr2, c0, c1, c2, *rest):
        wn = rest[:k]
        h_out = rest[k]
        gs = rest[k + 1:]
        W = wr0[...] + wr1[...] + wr2[...]
        bb = c0[...] + c1[...] + c2[...]
        m = (p0_ref[0, :, :] + p1_ref[0, :, :]
             + jnp.dot(h_ref[...], W, preferred_element_type=jnp.float32)
             + bb)
        hn = jnp.maximum(m, 0.0)
        h_out[...] = hn
        for i in range(k):
            gs[i][...] = jnp.dot(hn, wn[i][...],
                                 preferred_element_type=jnp.float32)

    in_specs = ([pl.BlockSpec((1, _TCB, _F), lambda i: (0, i, 0)),
                 pl.BlockSpec((1, _TCB, _F), lambda i: (1, i, 0)),
                 pl.BlockSpec((_TCB, _F), lambda i: (i, 0))]
                + [pl.BlockSpec((_F, _F), lambda i: (0, 0))] * 3
                + [pl.BlockSpec((1, _F), lambda i: (0, 0))] * 3
                + [pl.BlockSpec((_F, _F), lambda i: (0, 0))] * k)
    out_specs = [pl.BlockSpec((_TCB, _F), lambda i: (i, 0))] * (1 + k)
    out_shape = [jax.ShapeDtypeStruct((N, _F), jnp.float32)] * (1 + k)
    return pl.pallas_call(body, grid=(nb,), in_specs=in_specs,
                          out_specs=out_specs, out_shape=out_shape)(
        P, P, h, *Wroots, *brels, *Wnext)


def _final_call(P, h, Wroots, brels, Wh, bh):
    N = h.shape[0]
    nb = N // _TCB

    def body(p0_ref, p1_ref, h_ref, wr0, wr1, wr2, c0, c1, c2, wh, bhr, y_ref):
        W = wr0[...] + wr1[...] + wr2[...]
        bb = c0[...] + c1[...] + c2[...]
        m = (p0_ref[0, :, :] + p1_ref[0, :, :]
             + jnp.dot(h_ref[...], W, preferred_element_type=jnp.float32)
             + bb)
        hn = jnp.maximum(m, 0.0)
        y_ref[...] = jnp.dot(hn, wh[...],
                             preferred_element_type=jnp.float32) + bhr[...]

    in_specs = ([pl.BlockSpec((1, _TCB, _F), lambda i: (0, i, 0)),
                 pl.BlockSpec((1, _TCB, _F), lambda i: (1, i, 0)),
                 pl.BlockSpec((_TCB, _F), lambda i: (i, 0))]
                + [pl.BlockSpec((_F, _F), lambda i: (0, 0))] * 3
                + [pl.BlockSpec((1, _F), lambda i: (0, 0))] * 3
                + [pl.BlockSpec((_F, 1), lambda i: (0, 0)),
                   pl.BlockSpec((1, 1), lambda i: (0, 0))])
    out_specs = pl.BlockSpec((_TCB, 1), lambda i: (i, 0))
    out_shape = jax.ShapeDtypeStruct((N, 1), jnp.float32)
    return pl.pallas_call(body, grid=(nb,), in_specs=in_specs,
                          out_specs=out_specs, out_shape=out_shape)(
        P, P, h, *Wroots, *brels, Wh, bh)


def _prep_edges(ei, n_dst):
    E = ei.shape[1]
    nck = -(-E // _EPB)
    pad = nck * _EPB - E
    src = jnp.concatenate(
        [ei[0].astype(jnp.int32), jnp.zeros((pad,), jnp.int32)])
    dst = jnp.concatenate(
        [ei[1].astype(jnp.int32), jnp.full((pad,), n_dst, jnp.int32)])
    return src, dst, nck


def kernel(x_H, x_C, x_Others,
           edge_index_H_H, edge_index_H_C, edge_index_H_Others,
           edge_index_C_H, edge_index_C_C, edge_index_C_Others,
           edge_index_Others_H, edge_index_Others_C,
           edge_index_Others_Others, params):
    p = params
    x = {"H": x_H, "C": x_C, "Others": x_Others}
    edges = {
        ("H", "H"): edge_index_H_H,
        ("H", "C"): edge_index_H_C,
        ("H", "Others"): edge_index_H_Others,
        ("C", "H"): edge_index_C_H,
        ("C", "C"): edge_index_C_C,
        ("C", "Others"): edge_index_C_Others,
        ("Others", "H"): edge_index_Others_H,
        ("Others", "C"): edge_index_Others_C,
        ("Others", "Others"): edge_index_Others_Others,
    }
    n_node = {t: x[t].shape[0] for t in _TYPES}

    h0 = {}
    g0 = {}
    for t in _TYPES:
        wr = [p["conv0_%s_%s_Wrel" % (t, d)] for d in _TYPES]
        res = _enc_call(x[t], p["enc_%s_W1" % t],
                        p["enc_%s_b1" % t].reshape(1, -1),
                        p["enc_%s_W2" % t],
                        p["enc_%s_b2" % t].reshape(1, -1), wr)
        h0[t] = res[0]
        for i, d in enumerate(_TYPES):
            g0[(t, d)] = res[1 + i]

    rels0 = [(s, d) for (s, d) in edges]
    rel_meta0 = []
    sc_in0 = []
    for (s, d) in rels0:
        src, dst, nck = _prep_edges(edges[(s, d)], n_node[d])
        rel_meta0.append((s, d, n_node[s], nck))
        sc_in0.append((g0[(s, d)], src, dst))
    sc0 = _make_sc_scatter(rel_meta0, _TYPES, n_node)
    P0 = sc0(*[a[0] for a in sc_in0], *[a[1] for a in sc_in0],
             *[a[2] for a in sc_in0])
    P0 = {d: P0[i] for i, d in enumerate(_TYPES)}

    h1 = {}
    g1 = {}
    for d in _TYPES:
        wroots = [p["conv0_%s_%s_Wroot" % (s, d)] for s in _TYPES]
        brels = [p["conv0_%s_%s_brel" % (s, d)].reshape(1, -1)
                 for s in _TYPES]
        wnext = [p["conv1_%s_%s_Wrel" % (d, dd)] for dd in ("H", "C")]
        res = _comb_call(P0[d], h0[d], wroots, brels, wnext)
        h1[d] = res[0]
        for i, dd in enumerate(("H", "C")):
            g1[(d, dd)] = res[1 + i]

    rels1 = [(s, d) for (s, d) in edges if d != "Others"]
    rel_meta1 = []
    sc_in1 = []
    for (s, d) in rels1:
        src, dst, nck = _prep_edges(edges[(s, d)], n_node[d])
        rel_meta1.append((s, d, n_node[s], nck))
        sc_in1.append((g1[(s, d)], src, dst))
    sc1 = _make_sc_scatter(rel_meta1, ("H", "C"), n_node)
    P1 = sc1(*[a[0] for a in sc_in1], *[a[1] for a in sc_in1],
             *[a[2] for a in sc_in1])
    P1 = {d: P1[i] for i, d in enumerate(("H", "C"))}

    ys = {}
    for d in ("H", "C"):
        wroots = [p["conv1_%s_%s_Wroot" % (s, d)] for s in _TYPES]
        brels = [p["conv1_%s_%s_brel" % (s, d)].reshape(1, -1)
                 for s in _TYPES]
        ys[d] = _final_call(P1[d], h1[d], wroots, brels,
                            p["head_%s_W" % d],
                            p["head_%s_b" % d].reshape(1, 1))
    return ys["H"], ys["C"]


# speedup vs baseline: 6.4901x; 1.4939x over previous
import functools

import jax
import jax.numpy as jnp
from jax import lax
from jax.experimental import pallas as pl
from jax.experimental.pallas import tpu as pltpu
from jax.experimental.pallas import tpu_sc as plsc

_TYPES = ("H", "C", "Others")
_F = 16
_NC = 2
_NS = 16
_NW = _NC * _NS
_CHUNK = 128
_DEPTH = 4
_SEG = 32
_EPB = _CHUNK * _NW * _DEPTH
_ZROWS = 256


def _acc_rows(n):
    return ((n + 8 + 127) // 128) * 128


def _make_sc_scatter(rels, dsts, n_dst):
    mesh = plsc.VectorSubcoreMesh(core_axis_name="c", subcore_axis_name="s")
    out_type = [jax.ShapeDtypeStruct((2, _acc_rows(n_dst[d]), _F),
                                     jnp.float32)
                for d in dsts]
    scratch = ([pltpu.VMEM((_ZROWS, _F), jnp.float32),
                pltpu.VMEM((_SEG * _CHUNK,), jnp.int32),
                pltpu.VMEM((_SEG * _CHUNK,), jnp.int32),
                pltpu.VMEM((_DEPTH, _CHUNK, _F), jnp.float32)]
               + [pltpu.VMEM_SHARED((_acc_rows(n_dst[d]), _F), jnp.float32)
                  for d in dsts]
               + [pltpu.SemaphoreType.DMA, pltpu.SemaphoreType.DMA])
    R = len(rels)
    D = len(dsts)

    def body(*refs):
        g = refs[0:R]
        src = refs[R:2 * R]
        dst = refs[2 * R:3 * R]
        outs = refs[3 * R:3 * R + D]
        zbuf, sidx, didx, rows = refs[3 * R + D:3 * R + D + 4]
        accs = refs[3 * R + D + 4:3 * R + D + 4 + D]
        gsem, ssem = refs[-2], refs[-1]

        cid = lax.axis_index("c")
        sid = lax.axis_index("s")
        wid = sid * _NC + cid

        def _zb(i, c):
            zbuf[i, :] = jnp.zeros((_F,), jnp.float32)
            return c
        lax.fori_loop(0, _ZROWS, _zb, 0)
        for a_i, d in enumerate(dsts):
            per = _acc_rows(n_dst[d]) // _NS
            for off in range(0, per, _ZROWS):
                c = min(_ZROWS, per - off)
                pltpu.sync_copy(zbuf.at[pl.ds(0, c)],
                                accs[a_i].at[pl.ds(sid * per + off, c)])
        plsc.subcore_barrier()

        acc_of = {d: accs[i] for i, d in enumerate(dsts)}
        for r_i, (s_t, d_t, n_src, nck) in enumerate(rels):
            a = acc_of[d_t]
            m = nck * _CHUNK

            def _grp(j, c, r_i=r_i, a=a):
                gds = []
                for b in range(_DEPTH):
                    o = (j * _DEPTH + b) * _CHUNK
                    gds.append(pltpu.async_copy(
                        g[r_i].at[sidx.at[pl.ds(o, _CHUNK)]],
                        rows.at[b], gsem))
                for b in range(_DEPTH):
                    o = (j * _DEPTH + b) * _CHUNK
                    gds[b].wait()
                    pltpu.sync_copy(rows.at[b],
                                    a.at[didx.at[pl.ds(o, _CHUNK)]],
                                    add=True)
                return c

            for q in range(0, nck, _SEG):
                sn = min(_SEG, nck - q)
                e0 = wid * m + q * _CHUNK
                ne = sn * _CHUNK
                pltpu.sync_copy(src[r_i].at[pl.ds(e0, ne)],
                                sidx.at[pl.ds(0, ne)])
                pltpu.sync_copy(dst[r_i].at[pl.ds(e0, ne)],
                                didx.at[pl.ds(0, ne)])
                lax.fori_loop(0, sn // _DEPTH, _grp, 0)
        plsc.subcore_barrier()

        for a_i, d in enumerate(dsts):
            per = _acc_rows(n_dst[d]) // _NS
            for off in range(0, per, _ZROWS):
                c = min(_ZROWS, per - off)
                r0 = sid * per + off
                pltpu.sync_copy(accs[a_i].at[pl.ds(r0, c)],
                                zbuf.at[pl.ds(0, c)])
                pltpu.sync_copy(zbuf.at[pl.ds(0, c)],
                                outs[a_i].at[cid, pl.ds(r0, c)])

    return pl.kernel(body, out_type=out_type, mesh=mesh,
                     scratch_types=scratch,
                     compiler_params=pltpu.CompilerParams(
                         use_tc_tiling_on_sc=False))


_TCB = 2000


def _enc_call(x, W1, b1, W2, b2, Wrels):
    N = x.shape[0]
    k = len(Wrels)

    def body(x_ref, w1_ref, b1_ref, w2_ref, b2_ref, *rest):
        wr = rest[:k]
        h_ref = rest[k]
        gs = rest[k + 1:]
        z = jnp.maximum(
            jnp.dot(x_ref[...], w1_ref[...],
                    preferred_element_type=jnp.float32) + b1_ref[...], 0.0)
        h = jnp.maximum(
            jnp.dot(z, w2_ref[...],
                    preferred_element_type=jnp.float32) + b2_ref[...], 0.0)
        h_ref[...] = h
        for i in range(k):
            gs[i][...] = jnp.dot(h, wr[i][...],
                                 preferred_element_type=jnp.float32)

    in_specs = ([pl.BlockSpec((_TCB, 128), lambda i: (i, 0)),
                 pl.BlockSpec((128, 32), lambda i: (0, 0)),
                 pl.BlockSpec((1, 32), lambda i: (0, 0)),
                 pl.BlockSpec((32, _F), lambda i: (0, 0)),
                 pl.BlockSpec((1, _F), lambda i: (0, 0))]
                + [pl.BlockSpec((_F, _F), lambda i: (0, 0))] * k)
    out_specs = [pl.BlockSpec((_TCB, _F), lambda i: (i, 0))] * (1 + k)
    out_shape = [jax.ShapeDtypeStruct((N, _F), jnp.float32)] * (1 + k)
    return pl.pallas_call(body, grid=(N // _TCB,), in_specs=in_specs,
                          out_specs=out_specs, out_shape=out_shape)(
        x, W1, b1, W2, b2, *Wrels)


def _comb_call(P, h, Wroots, brels, Wnext):
    N = h.shape[0]
    nb = N // _TCB
    k = len(Wnext)

    def body(p0_ref, p1_ref, h_ref, wr0, wr1, wr2, c0, c1, c2, *rest):
        wn = rest[:k]
        h_out = rest[k]
        gs = rest[k + 1:]
        W = wr0[...] + wr1[...] + wr2[...]
        bb = c0[...] + c1[...] + c2[...]
        m = (p0_ref[0, :, :] + p1_ref[0, :, :]
             + jnp.dot(h_ref[...], W, preferred_element_type=jnp.float32)
             + bb)
        hn = jnp.maximum(m, 0.0)
        h_out[...] = hn
        for i in range(k):
            gs[i][...] = jnp.dot(hn, wn[i][...],
                                 preferred_element_type=jnp.float32)

    in_specs = ([pl.BlockSpec((1, _TCB, _F), lambda i: (0, i, 0)),
                 pl.BlockSpec((1, _TCB, _F), lambda i: (1, i, 0)),
                 pl.BlockSpec((_TCB, _F), lambda i: (i, 0))]
                + [pl.BlockSpec((_F, _F), lambda i: (0, 0))] * 3
                + [pl.BlockSpec((1, _F), lambda i: (0, 0))] * 3
                + [pl.BlockSpec((_F, _F), lambda i: (0, 0))] * k)
    out_specs = [pl.BlockSpec((_TCB, _F), lambda i: (i, 0))] * (1 + k)
    out_shape = [jax.ShapeDtypeStruct((N, _F), jnp.float32)] * (1 + k)
    return pl.pallas_call(body, grid=(nb,), in_specs=in_specs,
                          out_specs=out_specs, out_shape=out_shape)(
        P, P, h, *Wroots, *brels, *Wnext)


def _final_call(P, h, Wroots, brels, Wh, bh):
    N = h.shape[0]
    nb = N // _TCB

    def body(p0_ref, p1_ref, h_ref, wr0, wr1, wr2, c0, c1, c2, wh, bhr, y_ref):
        W = wr0[...] + wr1[...] + wr2[...]
        bb = c0[...] + c1[...] + c2[...]
        m = (p0_ref[0, :, :] + p1_ref[0, :, :]
             + jnp.dot(h_ref[...], W, preferred_element_type=jnp.float32)
             + bb)
        hn = jnp.maximum(m, 0.0)
        y_ref[...] = jnp.dot(hn, wh[...],
                             preferred_element_type=jnp.float32) + bhr[...]

    in_specs = ([pl.BlockSpec((1, _TCB, _F), lambda i: (0, i, 0)),
                 pl.BlockSpec((1, _TCB, _F), lambda i: (1, i, 0)),
                 pl.BlockSpec((_TCB, _F), lambda i: (i, 0))]
                + [pl.BlockSpec((_F, _F), lambda i: (0, 0))] * 3
                + [pl.BlockSpec((1, _F), lambda i: (0, 0))] * 3
                + [pl.BlockSpec((_F, 1), lambda i: (0, 0)),
                   pl.BlockSpec((1, 1), lambda i: (0, 0))])
    out_specs = pl.BlockSpec((_TCB, 1), lambda i: (i, 0))
    out_shape = jax.ShapeDtypeStruct((N, 1), jnp.float32)
    return pl.pallas_call(body, grid=(nb,), in_specs=in_specs,
                          out_specs=out_specs, out_shape=out_shape)(
        P, P, h, *Wroots, *brels, Wh, bh)


def _prep_edges(ei, n_dst):
    E = ei.shape[1]
    ng = -(-E // _EPB)
    nck = ng * _DEPTH
    pad = ng * _EPB - E
    src = jnp.concatenate(
        [ei[0].astype(jnp.int32), jnp.zeros((pad,), jnp.int32)])
    dst = jnp.concatenate(
        [ei[1].astype(jnp.int32), jnp.full((pad,), n_dst, jnp.int32)])
    return src, dst, nck


def kernel(x_H, x_C, x_Others,
           edge_index_H_H, edge_index_H_C, edge_index_H_Others,
           edge_index_C_H, edge_index_C_C, edge_index_C_Others,
           edge_index_Others_H, edge_index_Others_C,
           edge_index_Others_Others, params):
    p = params
    x = {"H": x_H, "C": x_C, "Others": x_Others}
    edges = {
        ("H", "H"): edge_index_H_H,
        ("H", "C"): edge_index_H_C,
        ("H", "Others"): edge_index_H_Others,
        ("C", "H"): edge_index_C_H,
        ("C", "C"): edge_index_C_C,
        ("C", "Others"): edge_index_C_Others,
        ("Others", "H"): edge_index_Others_H,
        ("Others", "C"): edge_index_Others_C,
        ("Others", "Others"): edge_index_Others_Others,
    }
    n_node = {t: x[t].shape[0] for t in _TYPES}

    h0 = {}
    g0 = {}
    for t in _TYPES:
        wr = [p["conv0_%s_%s_Wrel" % (t, d)] for d in _TYPES]
        res = _enc_call(x[t], p["enc_%s_W1" % t],
                        p["enc_%s_b1" % t].reshape(1, -1),
                        p["enc_%s_W2" % t],
                        p["enc_%s_b2" % t].reshape(1, -1), wr)
        h0[t] = res[0]
        for i, d in enumerate(_TYPES):
            g0[(t, d)] = res[1 + i]

    rels0 = [(s, d) for (s, d) in edges]
    rel_meta0 = []
    sc_in0 = []
    for (s, d) in rels0:
        src, dst, nck = _prep_edges(edges[(s, d)], n_node[d])
        rel_meta0.append((s, d, n_node[s], nck))
        sc_in0.append((g0[(s, d)], src, dst))
    sc0 = _make_sc_scatter(rel_meta0, _TYPES, n_node)
    P0 = sc0(*[a[0] for a in sc_in0], *[a[1] for a in sc_in0],
             *[a[2] for a in sc_in0])
    P0 = {d: P0[i] for i, d in enumerate(_TYPES)}

    h1 = {}
    g1 = {}
    for d in _TYPES:
        wroots = [p["conv0_%s_%s_Wroot" % (s, d)] for s in _TYPES]
        brels = [p["conv0_%s_%s_brel" % (s, d)].reshape(1, -1)
                 for s in _TYPES]
        wnext = [p["conv1_%s_%s_Wrel" % (d, dd)] for dd in ("H", "C")]
        res = _comb_call(P0[d], h0[d], wroots, brels, wnext)
        h1[d] = res[0]
        for i, dd in enumerate(("H", "C")):
            g1[(d, dd)] = res[1 + i]

    rels1 = [(s, d) for (s, d) in edges if d != "Others"]
    rel_meta1 = []
    sc_in1 = []
    for (s, d) in rels1:
        src, dst, nck = _prep_edges(edges[(s, d)], n_node[d])
        rel_meta1.append((s, d, n_node[s], nck))
        sc_in1.append((g1[(s, d)], src, dst))
    sc1 = _make_sc_scatter(rel_meta1, ("H", "C"), n_node)
    P1 = sc1(*[a[0] for a in sc_in1], *[a[1] for a in sc_in1],
             *[a[2] for a in sc_in1])
    P1 = {d: P1[i] for i, d in enumerate(("H", "C"))}

    ys = {}
    for d in ("H", "C"):
        wroots = [p["conv1_%s_%s_Wroot" % (s, d)] for s in _TYPES]
        brels = [p["conv1_%s_%s_brel" % (s, d)].reshape(1, -1)
                 for s in _TYPES]
        ys[d] = _final_call(P1[d], h1[d], wroots, brels,
                            p["head_%s_W" % d],
                            p["head_%s_b" % d].reshape(1, 1))
    return ys["H"], ys["C"]


# speedup vs baseline: 6.4956x; 1.0009x over previous
import functools

import jax
import jax.numpy as jnp
from jax import lax
from jax.experimental import pallas as pl
from jax.experimental.pallas import tpu as pltpu
from jax.experimental.pallas import tpu_sc as plsc

_TYPES = ("H", "C", "Others")
_F = 16
_NC = 2
_NS = 16
_NW = _NC * _NS
_CHUNK = 128
_DEPTH = 4
_SEG = 32
_EPB = _CHUNK * _NW * _DEPTH
_ZROWS = 256


def _acc_rows(n):
    return ((n + 8 + 127) // 128) * 128


def _make_sc_scatter(rels, dsts, n_dst):
    mesh = plsc.VectorSubcoreMesh(core_axis_name="c", subcore_axis_name="s")
    out_type = [jax.ShapeDtypeStruct((2, _acc_rows(n_dst[d]), _F),
                                     jnp.float32)
                for d in dsts]
    scratch = ([pltpu.VMEM((_ZROWS, _F), jnp.float32),
                pltpu.VMEM((_SEG * _CHUNK,), jnp.int32),
                pltpu.VMEM((_SEG * _CHUNK,), jnp.int32),
                pltpu.VMEM((_DEPTH, _CHUNK, _F), jnp.float32)]
               + [pltpu.VMEM_SHARED((_acc_rows(n_dst[d]), _F), jnp.float32)
                  for d in dsts]
               + [pltpu.SemaphoreType.DMA, pltpu.SemaphoreType.DMA])
    R = len(rels)
    D = len(dsts)

    def body(*refs):
        g = refs[0:R]
        src = refs[R:2 * R]
        dst = refs[2 * R:3 * R]
        outs = refs[3 * R:3 * R + D]
        zbuf, sidx, didx, rows = refs[3 * R + D:3 * R + D + 4]
        accs = refs[3 * R + D + 4:3 * R + D + 4 + D]
        gsem, ssem = refs[-2], refs[-1]

        cid = lax.axis_index("c")
        sid = lax.axis_index("s")
        wid = sid * _NC + cid

        def _zb(i, c):
            zbuf[i, :] = jnp.zeros((_F,), jnp.float32)
            return c
        lax.fori_loop(0, _ZROWS, _zb, 0)
        for a_i, d in enumerate(dsts):
            per = _acc_rows(n_dst[d]) // _NS
            for off in range(0, per, _ZROWS):
                c = min(_ZROWS, per - off)
                pltpu.sync_copy(zbuf.at[pl.ds(0, c)],
                                accs[a_i].at[pl.ds(sid * per + off, c)])
        plsc.subcore_barrier()

        acc_of = {d: accs[i] for i, d in enumerate(dsts)}
        for r_i, (s_t, d_t, n_src, nck) in enumerate(rels):
            a = acc_of[d_t]
            m = nck * _CHUNK

            def _grp(j, c, r_i=r_i, a=a):
                gds = []
                for b in range(_DEPTH):
                    o = (j * _DEPTH + b) * _CHUNK
                    gds.append(pltpu.async_copy(
                        g[r_i].at[sidx.at[pl.ds(o, _CHUNK)]],
                        rows.at[b], gsem))
                sds = []
                for b in range(_DEPTH):
                    o = (j * _DEPTH + b) * _CHUNK
                    gds[b].wait()
                    sds.append(pltpu.async_copy(
                        rows.at[b], a.at[didx.at[pl.ds(o, _CHUNK)]],
                        ssem, add=True))
                for b in range(_DEPTH):
                    sds[b].wait()
                return c

            for q in range(0, nck, _SEG):
                sn = min(_SEG, nck - q)
                e0 = wid * m + q * _CHUNK
                ne = sn * _CHUNK
                pltpu.sync_copy(src[r_i].at[pl.ds(e0, ne)],
                                sidx.at[pl.ds(0, ne)])
                pltpu.sync_copy(dst[r_i].at[pl.ds(e0, ne)],
                                didx.at[pl.ds(0, ne)])
                lax.fori_loop(0, sn // _DEPTH, _grp, 0)
        plsc.subcore_barrier()

        for a_i, d in enumerate(dsts):
            per = _acc_rows(n_dst[d]) // _NS
            for off in range(0, per, _ZROWS):
                c = min(_ZROWS, per - off)
                r0 = sid * per + off
                pltpu.sync_copy(accs[a_i].at[pl.ds(r0, c)],
                                zbuf.at[pl.ds(0, c)])
                pltpu.sync_copy(zbuf.at[pl.ds(0, c)],
                                outs[a_i].at[cid, pl.ds(r0, c)])

    return pl.kernel(body, out_type=out_type, mesh=mesh,
                     scratch_types=scratch,
                     compiler_params=pltpu.CompilerParams(
                         use_tc_tiling_on_sc=False))


_TCB = 2000


def _enc_call(x, W1, b1, W2, b2, Wrels):
    N = x.shape[0]
    k = len(Wrels)

    def body(x_ref, w1_ref, b1_ref, w2_ref, b2_ref, *rest):
        wr = rest[:k]
        h_ref = rest[k]
        gs = rest[k + 1:]
        z = jnp.maximum(
            jnp.dot(x_ref[...], w1_ref[...],
                    preferred_element_type=jnp.float32) + b1_ref[...], 0.0)
        h = jnp.maximum(
            jnp.dot(z, w2_ref[...],
                    preferred_element_type=jnp.float32) + b2_ref[...], 0.0)
        h_ref[...] = h
        for i in range(k):
            gs[i][...] = jnp.dot(h, wr[i][...],
                                 preferred_element_type=jnp.float32)

    in_specs = ([pl.BlockSpec((_TCB, 128), lambda i: (i, 0)),
                 pl.BlockSpec((128, 32), lambda i: (0, 0)),
                 pl.BlockSpec((1, 32), lambda i: (0, 0)),
                 pl.BlockSpec((32, _F), lambda i: (0, 0)),
                 pl.BlockSpec((1, _F), lambda i: (0, 0))]
                + [pl.BlockSpec((_F, _F), lambda i: (0, 0))] * k)
    out_specs = [pl.BlockSpec((_TCB, _F), lambda i: (i, 0))] * (1 + k)
    out_shape = [jax.ShapeDtypeStruct((N, _F), jnp.float32)] * (1 + k)
    return pl.pallas_call(body, grid=(N // _TCB,), in_specs=in_specs,
                          out_specs=out_specs, out_shape=out_shape)(
        x, W1, b1, W2, b2, *Wrels)


def _comb_call(P, h, Wroots, brels, Wnext):
    N = h.shape[0]
    nb = N // _TCB
    k = len(Wnext)

    def body(p0_ref, p1_ref, h_ref, wr0, wr1, wr2, c0, c1, c2, *rest):
        wn = rest[:k]
        h_out = rest[k]
        gs = rest[k + 1:]
        W = wr0[...] + wr1[...] + wr2[...]
        bb = c0[...] + c1[...] + c2[...]
        m = (p0_ref[0, :, :] + p1_ref[0, :, :]
             + jnp.dot(h_ref[...], W, preferred_element_type=jnp.float32)
             + bb)
        hn = jnp.maximum(m, 0.0)
        h_out[...] = hn
        for i in range(k):
            gs[i][...] = jnp.dot(hn, wn[i][...],
                                 preferred_element_type=jnp.float32)

    in_specs = ([pl.BlockSpec((1, _TCB, _F), lambda i: (0, i, 0)),
                 pl.BlockSpec((1, _TCB, _F), lambda i: (1, i, 0)),
                 pl.BlockSpec((_TCB, _F), lambda i: (i, 0))]
                + [pl.BlockSpec((_F, _F), lambda i: (0, 0))] * 3
                + [pl.BlockSpec((1, _F), lambda i: (0, 0))] * 3
                + [pl.BlockSpec((_F, _F), lambda i: (0, 0))] * k)
    out_specs = [pl.BlockSpec((_TCB, _F), lambda i: (i, 0))] * (1 + k)
    out_shape = [jax.ShapeDtypeStruct((N, _F), jnp.float32)] * (1 + k)
    return pl.pallas_call(body, grid=(nb,), in_specs=in_specs,
                          out_specs=out_specs, out_shape=out_shape)(
        P, P, h, *Wroots, *brels, *Wnext)


def _final_call(P, h, Wroots, brels, Wh, bh):
    N = h.shape[0]
    nb = N // _TCB

    def body(p0_ref, p1_ref, h_ref, wr0, wr1, wr2, c0, c1, c2, wh, bhr, y_ref):
        W = wr0[...] + wr1[...] + wr2[...]
        bb = c0[...] + c1[...] + c2[...]
        m = (p0_ref[0, :, :] + p1_ref[0, :, :]
             + jnp.dot(h_ref[...], W, preferred_element_type=jnp.float32)
             + bb)
        hn = jnp.maximum(m, 0.0)
        y_ref[...] = jnp.dot(hn, wh[...],
                             preferred_element_type=jnp.float32) + bhr[...]

    in_specs = ([pl.BlockSpec((1, _TCB, _F), lambda i: (0, i, 0)),
                 pl.BlockSpec((1, _TCB, _F), lambda i: (1, i, 0)),
                 pl.BlockSpec((_TCB, _F), lambda i: (i, 0))]
                + [pl.BlockSpec((_F, _F), lambda i: (0, 0))] * 3
                + [pl.BlockSpec((1, _F), lambda i: (0, 0))] * 3
                + [pl.BlockSpec((_F, 1), lambda i: (0, 0)),
                   pl.BlockSpec((1, 1), lambda i: (0, 0))])
    out_specs = pl.BlockSpec((_TCB, 1), lambda i: (i, 0))
    out_shape = jax.ShapeDtypeStruct((N, 1), jnp.float32)
    return pl.pallas_call(body, grid=(nb,), in_specs=in_specs,
                          out_specs=out_specs, out_shape=out_shape)(
        P, P, h, *Wroots, *brels, Wh, bh)


def _prep_edges(ei, n_dst):
    E = ei.shape[1]
    ng = -(-E // _EPB)
    nck = ng * _DEPTH
    pad = ng * _EPB - E
    src = jnp.concatenate(
        [ei[0].astype(jnp.int32), jnp.zeros((pad,), jnp.int32)])
    dst = jnp.concatenate(
        [ei[1].astype(jnp.int32), jnp.full((pad,), n_dst, jnp.int32)])
    return src, dst, nck


def kernel(x_H, x_C, x_Others,
           edge_index_H_H, edge_index_H_C, edge_index_H_Others,
           edge_index_C_H, edge_index_C_C, edge_index_C_Others,
           edge_index_Others_H, edge_index_Others_C,
           edge_index_Others_Others, params):
    p = params
    x = {"H": x_H, "C": x_C, "Others": x_Others}
    edges = {
        ("H", "H"): edge_index_H_H,
        ("H", "C"): edge_index_H_C,
        ("H", "Others"): edge_index_H_Others,
        ("C", "H"): edge_index_C_H,
        ("C", "C"): edge_index_C_C,
        ("C", "Others"): edge_index_C_Others,
        ("Others", "H"): edge_index_Others_H,
        ("Others", "C"): edge_index_Others_C,
        ("Others", "Others"): edge_index_Others_Others,
    }
    n_node = {t: x[t].shape[0] for t in _TYPES}

    h0 = {}
    g0 = {}
    for t in _TYPES:
        wr = [p["conv0_%s_%s_Wrel" % (t, d)] for d in _TYPES]
        res = _enc_call(x[t], p["enc_%s_W1" % t],
                        p["enc_%s_b1" % t].reshape(1, -1),
                        p["enc_%s_W2" % t],
                        p["enc_%s_b2" % t].reshape(1, -1), wr)
        h0[t] = res[0]
        for i, d in enumerate(_TYPES):
            g0[(t, d)] = res[1 + i]

    rels0 = [(s, d) for (s, d) in edges]
    rel_meta0 = []
    sc_in0 = []
    for (s, d) in rels0:
        src, dst, nck = _prep_edges(edges[(s, d)], n_node[d])
        rel_meta0.append((s, d, n_node[s], nck))
        sc_in0.append((g0[(s, d)], src, dst))
    sc0 = _make_sc_scatter(rel_meta0, _TYPES, n_node)
    P0 = sc0(*[a[0] for a in sc_in0], *[a[1] for a in sc_in0],
             *[a[2] for a in sc_in0])
    P0 = {d: P0[i] for i, d in enumerate(_TYPES)}

    h1 = {}
    g1 = {}
    for d in _TYPES:
        wroots = [p["conv0_%s_%s_Wroot" % (s, d)] for s in _TYPES]
        brels = [p["conv0_%s_%s_brel" % (s, d)].reshape(1, -1)
                 for s in _TYPES]
        wnext = [p["conv1_%s_%s_Wrel" % (d, dd)] for dd in ("H", "C")]
        res = _comb_call(P0[d], h0[d], wroots, brels, wnext)
        h1[d] = res[0]
        for i, dd in enumerate(("H", "C")):
            g1[(d, dd)] = res[1 + i]

    rels1 = [(s, d) for (s, d) in edges if d != "Others"]
    rel_meta1 = []
    sc_in1 = []
    for (s, d) in rels1:
        src, dst, nck = _prep_edges(edges[(s, d)], n_node[d])
        rel_meta1.append((s, d, n_node[s], nck))
        sc_in1.append((g1[(s, d)], src, dst))
    sc1 = _make_sc_scatter(rel_meta1, ("H", "C"), n_node)
    P1 = sc1(*[a[0] for a in sc_in1], *[a[1] for a in sc_in1],
             *[a[2] for a in sc_in1])
    P1 = {d: P1[i] for i, d in enumerate(("H", "C"))}

    ys = {}
    for d in ("H", "C"):
        wroots = [p["conv1_%s_%s_Wroot" % (s, d)] for s in _TYPES]
        brels = [p["conv1_%s_%s_brel" % (s, d)].reshape(1, -1)
                 for s in _TYPES]
        ys[d] = _final_call(P1[d], h1[d], wroots, brels,
                            p["head_%s_W" % d],
                            p["head_%s_b" % d].reshape(1, 1))
    return ys["H"], ys["C"]


# speedup vs baseline: 6.6949x; 1.0307x over previous
import functools

import jax
import jax.numpy as jnp
from jax import lax
from jax.experimental import pallas as pl
from jax.experimental.pallas import tpu as pltpu
from jax.experimental.pallas import tpu_sc as plsc

_TYPES = ("H", "C", "Others")
_F = 16
_NC = 2
_NS = 16
_NW = _NC * _NS
_CHUNK = 128
_DEPTH = 8
_SEG = 32
_EPB = _CHUNK * _NW * 4
_ZROWS = 256


def _acc_rows(n):
    return ((n + 8 + 127) // 128) * 128


def _make_sc_scatter(nck, acc_total, outs_meta):
    mesh = plsc.VectorSubcoreMesh(core_axis_name="c", subcore_axis_name="s")
    out_type = [jax.ShapeDtypeStruct((2, r, _F), jnp.float32)
                for (_, r) in outs_meta]
    scratch = [pltpu.VMEM((_ZROWS, _F), jnp.float32),
               pltpu.VMEM((_SEG * _CHUNK,), jnp.int32),
               pltpu.VMEM((_SEG * _CHUNK,), jnp.int32),
               pltpu.VMEM((_DEPTH, _CHUNK, _F), jnp.float32),
               pltpu.VMEM_SHARED((acc_total, _F), jnp.float32),
               pltpu.SemaphoreType.DMA, pltpu.SemaphoreType.DMA]
    D = len(outs_meta)

    def body(g, srcx, dstx, *refs):
        outs = refs[0:D]
        zbuf, sidx, didx, rows, acc, gsem, ssem = refs[D:]

        cid = lax.axis_index("c")
        sid = lax.axis_index("s")
        wid = sid * _NC + cid
        m = nck * _CHUNK

        def _zb(i, c):
            zbuf[i, :] = jnp.zeros((_F,), jnp.float32)
            return c
        lax.fori_loop(0, _ZROWS, _zb, 0)
        per = acc_total // _NS
        for off in range(0, per, _ZROWS):
            c = min(_ZROWS, per - off)
            pltpu.sync_copy(zbuf.at[pl.ds(0, c)],
                            acc.at[pl.ds(sid * per + off, c)])
        plsc.subcore_barrier()

        def _do_group(j0, depth):
            gds = []
            for b in range(depth):
                o = (j0 + b) * _CHUNK
                gds.append(pltpu.async_copy(
                    g.at[sidx.at[pl.ds(o, _CHUNK)]], rows.at[b], gsem))
            sds = []
            for b in range(depth):
                o = (j0 + b) * _CHUNK
                gds[b].wait()
                sds.append(pltpu.async_copy(
                    rows.at[b], acc.at[didx.at[pl.ds(o, _CHUNK)]],
                    ssem, add=True))
            for b in range(depth):
                sds[b].wait()

        def _do_seg(e0, sn):
            ne = sn * _CHUNK
            pltpu.sync_copy(srcx.at[pl.ds(e0, ne)], sidx.at[pl.ds(0, ne)])
            pltpu.sync_copy(dstx.at[pl.ds(e0, ne)], didx.at[pl.ds(0, ne)])
            full = sn // _DEPTH
            tail = sn - full * _DEPTH
            if full:
                lax.fori_loop(
                    0, full,
                    lambda j, c: (_do_group(j * _DEPTH, _DEPTH), c)[1], 0)
            if tail:
                _do_group(full * _DEPTH, tail)

        n_full = nck // _SEG
        tail_ck = nck - n_full * _SEG

        def _seg(si, c):
            _do_seg(wid * m + si * _SEG * _CHUNK, _SEG)
            return c
        lax.fori_loop(0, n_full, _seg, 0)
        if tail_ck:
            _do_seg(wid * m + n_full * _SEG * _CHUNK, tail_ck)
        plsc.subcore_barrier()

        for o_i, (a0, r) in enumerate(outs_meta):
            per_d = r // _NS
            for off in range(0, per_d, _ZROWS):
                c = min(_ZROWS, per_d - off)
                r0 = sid * per_d + off
                pltpu.sync_copy(acc.at[pl.ds(a0 + r0, c)],
                                zbuf.at[pl.ds(0, c)])
                pltpu.sync_copy(zbuf.at[pl.ds(0, c)],
                                outs[o_i].at[cid, pl.ds(r0, c)])

    return pl.kernel(body, out_type=out_type, mesh=mesh,
                     scratch_types=scratch,
                     compiler_params=pltpu.CompilerParams(
                         use_tc_tiling_on_sc=False))


_TCB = 2000


def _enc_call(x, W1, b1, W2, b2, Wrels):
    N = x.shape[0]
    k = len(Wrels)

    def body(x_ref, w1_ref, b1_ref, w2_ref, b2_ref, *rest):
        wr = rest[:k]
        h_ref = rest[k]
        gs = rest[k + 1:]
        z = jnp.maximum(
            jnp.dot(x_ref[...], w1_ref[...],
                    preferred_element_type=jnp.float32) + b1_ref[...], 0.0)
        h = jnp.maximum(
            jnp.dot(z, w2_ref[...],
                    preferred_element_type=jnp.float32) + b2_ref[...], 0.0)
        h_ref[...] = h
        for i in range(k):
            gs[i][...] = jnp.dot(h, wr[i][...],
                                 preferred_element_type=jnp.float32)

    in_specs = ([pl.BlockSpec((_TCB, 128), lambda i: (i, 0)),
                 pl.BlockSpec((128, 32), lambda i: (0, 0)),
                 pl.BlockSpec((1, 32), lambda i: (0, 0)),
                 pl.BlockSpec((32, _F), lambda i: (0, 0)),
                 pl.BlockSpec((1, _F), lambda i: (0, 0))]
                + [pl.BlockSpec((_F, _F), lambda i: (0, 0))] * k)
    out_specs = [pl.BlockSpec((_TCB, _F), lambda i: (i, 0))] * (1 + k)
    out_shape = [jax.ShapeDtypeStruct((N, _F), jnp.float32)] * (1 + k)
    return pl.pallas_call(body, grid=(N // _TCB,), in_specs=in_specs,
                          out_specs=out_specs, out_shape=out_shape)(
        x, W1, b1, W2, b2, *Wrels)


def _comb_call(P, h, Wroots, brels, Wnext):
    N = h.shape[0]
    nb = N // _TCB
    k = len(Wnext)

    def body(p0_ref, p1_ref, h_ref, wr0, wr1, wr2, c0, c1, c2, *rest):
        wn = rest[:k]
        h_out = rest[k]
        gs = rest[k + 1:]
        W = wr0[...] + wr1[...] + wr2[...]
        bb = c0[...] + c1[...] + c2[...]
        m = (p0_ref[0, :, :] + p1_ref[0, :, :]
             + jnp.dot(h_ref[...], W, preferred_element_type=jnp.float32)
             + bb)
        hn = jnp.maximum(m, 0.0)
        h_out[...] = hn
        for i in range(k):
            gs[i][...] = jnp.dot(hn, wn[i][...],
                                 preferred_element_type=jnp.float32)

    in_specs = ([pl.BlockSpec((1, _TCB, _F), lambda i: (0, i, 0)),
                 pl.BlockSpec((1, _TCB, _F), lambda i: (1, i, 0)),
                 pl.BlockSpec((_TCB, _F), lambda i: (i, 0))]
                + [pl.BlockSpec((_F, _F), lambda i: (0, 0))] * 3
                + [pl.BlockSpec((1, _F), lambda i: (0, 0))] * 3
                + [pl.BlockSpec((_F, _F), lambda i: (0, 0))] * k)
    out_specs = [pl.BlockSpec((_TCB, _F), lambda i: (i, 0))] * (1 + k)
    out_shape = [jax.ShapeDtypeStruct((N, _F), jnp.float32)] * (1 + k)
    return pl.pallas_call(body, grid=(nb,), in_specs=in_specs,
                          out_specs=out_specs, out_shape=out_shape)(
        P, P, h, *Wroots, *brels, *Wnext)


def _final_call(P, h, Wroots, brels, Wh, bh):
    N = h.shape[0]
    nb = N // _TCB

    def body(p0_ref, p1_ref, h_ref, wr0, wr1, wr2, c0, c1, c2, wh, bhr, y_ref):
        W = wr0[...] + wr1[...] + wr2[...]
        bb = c0[...] + c1[...] + c2[...]
        m = (p0_ref[0, :, :] + p1_ref[0, :, :]
             + jnp.dot(h_ref[...], W, preferred_element_type=jnp.float32)
             + bb)
        hn = jnp.maximum(m, 0.0)
        y_ref[...] = jnp.dot(hn, wh[...],
                             preferred_element_type=jnp.float32) + bhr[...]

    in_specs = ([pl.BlockSpec((1, _TCB, _F), lambda i: (0, i, 0)),
                 pl.BlockSpec((1, _TCB, _F), lambda i: (1, i, 0)),
                 pl.BlockSpec((_TCB, _F), lambda i: (i, 0))]
                + [pl.BlockSpec((_F, _F), lambda i: (0, 0))] * 3
                + [pl.BlockSpec((1, _F), lambda i: (0, 0))] * 3
                + [pl.BlockSpec((_F, 1), lambda i: (0, 0)),
                   pl.BlockSpec((1, 1), lambda i: (0, 0))])
    out_specs = pl.BlockSpec((_TCB, 1), lambda i: (i, 0))
    out_shape = jax.ShapeDtypeStruct((N, 1), jnp.float32)
    return pl.pallas_call(body, grid=(nb,), in_specs=in_specs,
                          out_specs=out_specs, out_shape=out_shape)(
        P, P, h, *Wroots, *brels, Wh, bh)


def _prep_layer(rel_list, g_dict, edges, n_node, dsts):
    acc_off = {}
    off = 0
    for d in dsts:
        acc_off[d] = off
        off += _acc_rows(n_node[d])
    acc_total = off
    g_parts, src_parts, dst_parts = [], [], []
    gbase = 0
    for (s, d) in rel_list:
        ei = edges[(s, d)]
        g_parts.append(g_dict[(s, d)])
        src_parts.append(ei[0].astype(jnp.int32) + gbase)
        dst_parts.append(ei[1].astype(jnp.int32) + acc_off[d])
        gbase += n_node[s]
    E = sum(edges[r].shape[1] for r in rel_list)
    ng = -(-E // _EPB)
    nck = ng * 4
    pad = ng * _EPB - E
    trash = acc_off[dsts[0]] + n_node[dsts[0]]
    src_cat = jnp.concatenate(src_parts + [jnp.zeros((pad,), jnp.int32)])
    dst_cat = jnp.concatenate(dst_parts
                              + [jnp.full((pad,), trash, jnp.int32)])
    g_cat = jnp.concatenate(g_parts, axis=0)
    outs_meta = [(acc_off[d], _acc_rows(n_node[d])) for d in dsts]
    return g_cat, src_cat, dst_cat, nck, acc_total, outs_meta


def kernel(x_H, x_C, x_Others,
           edge_index_H_H, edge_index_H_C, edge_index_H_Others,
           edge_index_C_H, edge_index_C_C, edge_index_C_Others,
           edge_index_Others_H, edge_index_Others_C,
           edge_index_Others_Others, params):
    p = params
    x = {"H": x_H, "C": x_C, "Others": x_Others}
    edges = {
        ("H", "H"): edge_index_H_H,
        ("H", "C"): edge_index_H_C,
        ("H", "Others"): edge_index_H_Others,
        ("C", "H"): edge_index_C_H,
        ("C", "C"): edge_index_C_C,
        ("C", "Others"): edge_index_C_Others,
        ("Others", "H"): edge_index_Others_H,
        ("Others", "C"): edge_index_Others_C,
        ("Others", "Others"): edge_index_Others_Others,
    }
    n_node = {t: x[t].shape[0] for t in _TYPES}

    h0 = {}
    g0 = {}
    for t in _TYPES:
        wr = [p["conv0_%s_%s_Wrel" % (t, d)] for d in _TYPES]
        res = _enc_call(x[t], p["enc_%s_W1" % t],
                        p["enc_%s_b1" % t].reshape(1, -1),
                        p["enc_%s_W2" % t],
                        p["enc_%s_b2" % t].reshape(1, -1), wr)
        h0[t] = res[0]
        for i, d in enumerate(_TYPES):
            g0[(t, d)] = res[1 + i]

    rels0 = [(s, d) for (s, d) in edges]
    g_cat, src_cat, dst_cat, nck, acc_total, meta = _prep_layer(
        rels0, g0, edges, n_node, _TYPES)
    sc0 = _make_sc_scatter(nck, acc_total, meta)
    P0 = sc0(g_cat, src_cat, dst_cat)
    P0 = {d: P0[i] for i, d in enumerate(_TYPES)}

    h1 = {}
    g1 = {}
    for d in _TYPES:
        wroots = [p["conv0_%s_%s_Wroot" % (s, d)] for s in _TYPES]
        brels = [p["conv0_%s_%s_brel" % (s, d)].reshape(1, -1)
                 for s in _TYPES]
        wnext = [p["conv1_%s_%s_Wrel" % (d, dd)] for dd in ("H", "C")]
        res = _comb_call(P0[d], h0[d], wroots, brels, wnext)
        h1[d] = res[0]
        for i, dd in enumerate(("H", "C")):
            g1[(d, dd)] = res[1 + i]

    rels1 = [(s, d) for (s, d) in edges if d != "Others"]
    g_cat1, src_cat1, dst_cat1, nck1, acc_total1, meta1 = _prep_layer(
        rels1, g1, edges, n_node, ("H", "C"))
    sc1 = _make_sc_scatter(nck1, acc_total1, meta1)
    P1 = sc1(g_cat1, src_cat1, dst_cat1)
    P1 = {d: P1[i] for i, d in enumerate(("H", "C"))}

    ys = {}
    for d in ("H", "C"):
        wroots = [p["conv1_%s_%s_Wroot" % (s, d)] for s in _TYPES]
        brels = [p["conv1_%s_%s_brel" % (s, d)].reshape(1, -1)
                 for s in _TYPES]
        ys[d] = _final_call(P1[d], h1[d], wroots, brels,
                            p["head_%s_W" % d],
                            p["head_%s_b" % d].reshape(1, 1))
    return ys["H"], ys["C"]


# speedup vs baseline: 11.0166x; 1.6455x over previous
import functools

import jax
import jax.numpy as jnp
from jax import lax
from jax.experimental import pallas as pl
from jax.experimental.pallas import tpu as pltpu
from jax.experimental.pallas import tpu_sc as plsc

_TYPES = ("H", "C", "Others")
_F = 16
_NC = 2
_NS = 16
_NW = _NC * _NS
_CHUNK = 128
_DEPTH = 8
_SEG = 32
_EPB = _CHUNK * _NW * 4
_ZROWS = 256


def _acc_rows(n):
    return ((n + 8 + 127) // 128) * 128


def _make_sc_scatter(nck, acc_total, outs_meta):
    mesh = plsc.VectorSubcoreMesh(core_axis_name="c", subcore_axis_name="s")
    out_type = [jax.ShapeDtypeStruct((2, r, _F), jnp.float32)
                for (_, r) in outs_meta]
    scratch = [pltpu.VMEM((_ZROWS, _F), jnp.float32),
               pltpu.VMEM((_SEG * _CHUNK,), jnp.int32),
               pltpu.VMEM((_SEG * _CHUNK,), jnp.int32),
               pltpu.VMEM((_DEPTH, _CHUNK, _F), jnp.float32),
               pltpu.VMEM_SHARED((acc_total, _F), jnp.float32),
               pltpu.SemaphoreType.DMA, pltpu.SemaphoreType.DMA]
    D = len(outs_meta)

    def body(g, srcx, dstx, *refs):
        outs = refs[0:D]
        zbuf, sidx, didx, rows, acc, gsem, ssem = refs[D:]

        cid = lax.axis_index("c")
        sid = lax.axis_index("s")
        wid = sid * _NC + cid
        m = nck * _CHUNK

        def _zb(i, c):
            zbuf[i, :] = jnp.zeros((_F,), jnp.float32)
            return c
        lax.fori_loop(0, _ZROWS, _zb, 0)
        per = acc_total // _NS
        for off in range(0, per, _ZROWS):
            c = min(_ZROWS, per - off)
            pltpu.sync_copy(zbuf.at[pl.ds(0, c)],
                            acc.at[pl.ds(sid * per + off, c)])
        plsc.subcore_barrier()

        def _do_group(j0, depth):
            gds = []
            for b in range(depth):
                o = (j0 + b) * _CHUNK
                gds.append(pltpu.async_copy(
                    g.at[sidx.at[pl.ds(o, _CHUNK)]], rows.at[b], gsem))
            sds = []
            for b in range(depth):
                o = (j0 + b) * _CHUNK
                gds[b].wait()
                sds.append(pltpu.async_copy(
                    rows.at[b], acc.at[didx.at[pl.ds(o, _CHUNK)]],
                    ssem, add=True))
            for b in range(depth):
                sds[b].wait()

        def _do_seg(e0, sn):
            ne = sn * _CHUNK
            pltpu.sync_copy(srcx.at[pl.ds(e0, ne)], sidx.at[pl.ds(0, ne)])
            pltpu.sync_copy(dstx.at[pl.ds(e0, ne)], didx.at[pl.ds(0, ne)])
            full = sn // _DEPTH
            tail = sn - full * _DEPTH
            if full:
                lax.fori_loop(
                    0, full,
                    lambda j, c: (_do_group(j * _DEPTH, _DEPTH), c)[1], 0)
            if tail:
                _do_group(full * _DEPTH, tail)

        n_full = nck // _SEG
        tail_ck = nck - n_full * _SEG

        def _seg(si, c):
            _do_seg(wid * m + si * _SEG * _CHUNK, _SEG)
            return c
        lax.fori_loop(0, n_full, _seg, 0)
        if tail_ck:
            _do_seg(wid * m + n_full * _SEG * _CHUNK, tail_ck)
        plsc.subcore_barrier()

        for o_i, (a0, r) in enumerate(outs_meta):
            per_d = r // _NS
            for off in range(0, per_d, _ZROWS):
                c = min(_ZROWS, per_d - off)
                r0 = sid * per_d + off
                pltpu.sync_copy(acc.at[pl.ds(a0 + r0, c)],
                                zbuf.at[pl.ds(0, c)])
                pltpu.sync_copy(zbuf.at[pl.ds(0, c)],
                                outs[o_i].at[cid, pl.ds(r0, c)])

    return pl.kernel(body, out_type=out_type, mesh=mesh,
                     scratch_types=scratch,
                     compiler_params=pltpu.CompilerParams(
                         use_tc_tiling_on_sc=False))


_TPB = 256
_PK = 8


def _bd8(W):
    return jnp.kron(jnp.eye(_PK, dtype=W.dtype), W)


def _t8(b):
    return jnp.tile(b.reshape(-1), _PK).reshape(1, -1)


def _enc_call(xp, W1, b1, W2, b2, Wrels):
    M = xp.shape[0]
    k = len(Wrels)

    def body(x_ref, w1_ref, b1_ref, w2_ref, b2_ref, *rest):
        wr = rest[:k]
        h_ref = rest[k]
        gs = rest[k + 1:]
        z = jnp.maximum(
            jnp.dot(x_ref[...], w1_ref[...],
                    preferred_element_type=jnp.float32) + b1_ref[...], 0.0)
        h = jnp.maximum(
            jnp.dot(z, w2_ref[...],
                    preferred_element_type=jnp.float32) + b2_ref[...], 0.0)
        h_ref[...] = h
        for i in range(k):
            gs[i][...] = jnp.dot(h, wr[i][...],
                                 preferred_element_type=jnp.float32)

    in_specs = ([pl.BlockSpec((_TPB, 1024), lambda i: (i, 0)),
                 pl.BlockSpec((1024, 256), lambda i: (0, 0)),
                 pl.BlockSpec((1, 256), lambda i: (0, 0)),
                 pl.BlockSpec((256, 128), lambda i: (0, 0)),
                 pl.BlockSpec((1, 128), lambda i: (0, 0))]
                + [pl.BlockSpec((128, 128), lambda i: (0, 0))] * k)
    out_specs = [pl.BlockSpec((_TPB, 128), lambda i: (i, 0))] * (1 + k)
    out_shape = [jax.ShapeDtypeStruct((M, 128), jnp.float32)] * (1 + k)
    return pl.pallas_call(body, grid=(-(-M // _TPB),), in_specs=in_specs,
                          out_specs=out_specs, out_shape=out_shape)(
        xp, W1, b1, W2, b2, *Wrels)


def _comb_call(P, h, Wroots, brels, Wnext, Wh=None, bh=None):
    M = h.shape[0]
    nb = -(-M // _TPB)
    k = len(Wnext)
    head = Wh is not None

    def body(p0_ref, p1_ref, h_ref, wr0, wr1, wr2, c0, c1, c2, *rest):
        wn = rest[:k]
        rest = rest[k:]
        if head:
            wh_ref, bh_ref = rest[0], rest[1]
            rest = rest[2:]
        h_out = rest[0]
        gs = rest[1:1 + k]
        W = wr0[...] + wr1[...] + wr2[...]
        bb = c0[...] + c1[...] + c2[...]
        m = (p0_ref[0, :, :] + p1_ref[0, :, :]
             + jnp.dot(h_ref[...], W, preferred_element_type=jnp.float32)
             + bb)
        hn = jnp.maximum(m, 0.0)
        h_out[...] = hn
        for i in range(k):
            gs[i][...] = jnp.dot(hn, wn[i][...],
                                 preferred_element_type=jnp.float32)
        if head:
            y_ref = rest[1 + k]
            y_ref[...] = jnp.dot(hn, wh_ref[...],
                                 preferred_element_type=jnp.float32) \
                + bh_ref[...]

    in_specs = ([pl.BlockSpec((1, _TPB, 128), lambda i: (0, i, 0)),
                 pl.BlockSpec((1, _TPB, 128), lambda i: (1, i, 0)),
                 pl.BlockSpec((_TPB, 128), lambda i: (i, 0))]
                + [pl.BlockSpec((128, 128), lambda i: (0, 0))] * 3
                + [pl.BlockSpec((1, 128), lambda i: (0, 0))] * 3
                + [pl.BlockSpec((128, 128), lambda i: (0, 0))] * k)
    out_specs = [pl.BlockSpec((_TPB, 128), lambda i: (i, 0))] * (1 + k)
    out_shape = [jax.ShapeDtypeStruct((M, 128), jnp.float32)] * (1 + k)
    args = [P, P, h] + list(Wroots) + list(brels) + list(Wnext)
    if head:
        in_specs += [pl.BlockSpec((128, _PK), lambda i: (0, 0)),
                     pl.BlockSpec((1, _PK), lambda i: (0, 0))]
        out_specs = out_specs + [pl.BlockSpec((_TPB, _PK),
                                              lambda i: (i, 0))]
        out_shape = out_shape + [jax.ShapeDtypeStruct((M, _PK),
                                                      jnp.float32)]
        args += [Wh, bh]
    return pl.pallas_call(body, grid=(nb,), in_specs=in_specs,
                          out_specs=out_specs, out_shape=out_shape)(*args)


def _prep_layer(rel_list, g_dict, edges, n_node, dsts):
    acc_off = {}
    off = 0
    for d in dsts:
        acc_off[d] = off
        off += _acc_rows(n_node[d])
    acc_total = off
    g_parts, src_parts, dst_parts = [], [], []
    gbase = 0
    for (s, d) in rel_list:
        ei = edges[(s, d)]
        g_parts.append(g_dict[(s, d)])
        src_parts.append(ei[0].astype(jnp.int32) + gbase)
        dst_parts.append(ei[1].astype(jnp.int32) + acc_off[d])
        gbase += n_node[s]
    E = sum(edges[r].shape[1] for r in rel_list)
    ng = -(-E // _EPB)
    nck = ng * 4
    pad = ng * _EPB - E
    trash = acc_off[dsts[0]] + n_node[dsts[0]]
    src_cat = jnp.concatenate(src_parts + [jnp.zeros((pad,), jnp.int32)])
    dst_cat = jnp.concatenate(dst_parts
                              + [jnp.full((pad,), trash, jnp.int32)])
    g_cat = jnp.concatenate(g_parts, axis=0).reshape(-1, _F)
    outs_meta = [(acc_off[d], _acc_rows(n_node[d])) for d in dsts]
    return g_cat, src_cat, dst_cat, nck, acc_total, outs_meta


def kernel(x_H, x_C, x_Others,
           edge_index_H_H, edge_index_H_C, edge_index_H_Others,
           edge_index_C_H, edge_index_C_C, edge_index_C_Others,
           edge_index_Others_H, edge_index_Others_C,
           edge_index_Others_Others, params):
    p = params
    x = {"H": x_H, "C": x_C, "Others": x_Others}
    edges = {
        ("H", "H"): edge_index_H_H,
        ("H", "C"): edge_index_H_C,
        ("H", "Others"): edge_index_H_Others,
        ("C", "H"): edge_index_C_H,
        ("C", "C"): edge_index_C_C,
        ("C", "Others"): edge_index_C_Others,
        ("Others", "H"): edge_index_Others_H,
        ("Others", "C"): edge_index_Others_C,
        ("Others", "Others"): edge_index_Others_Others,
    }
    n_node = {t: x[t].shape[0] for t in _TYPES}

    h0 = {}
    g0 = {}
    for t in _TYPES:
        wr = [_bd8(p["conv0_%s_%s_Wrel" % (t, d)]) for d in _TYPES]
        res = _enc_call(x[t].reshape(-1, 128 * _PK),
                        _bd8(p["enc_%s_W1" % t]), _t8(p["enc_%s_b1" % t]),
                        _bd8(p["enc_%s_W2" % t]), _t8(p["enc_%s_b2" % t]),
                        wr)
        h0[t] = res[0]
        for i, d in enumerate(_TYPES):
            g0[(t, d)] = res[1 + i]

    rels0 = [(s, d) for (s, d) in edges]
    g_cat, src_cat, dst_cat, nck, acc_total, meta = _prep_layer(
        rels0, g0, edges, n_node, _TYPES)
    sc0 = _make_sc_scatter(nck, acc_total, meta)
    P0 = sc0(g_cat, src_cat, dst_cat)
    P0 = {d: P0[i] for i, d in enumerate(_TYPES)}

    h1 = {}
    g1 = {}
    for d in _TYPES:
        wroots = [_bd8(p["conv0_%s_%s_Wroot" % (s, d)]) for s in _TYPES]
        brels = [_t8(p["conv0_%s_%s_brel" % (s, d)]) for s in _TYPES]
        wnext = [_bd8(p["conv1_%s_%s_Wrel" % (d, dd)])
                 for dd in ("H", "C")]
        res = _comb_call(P0[d].reshape(2, -1, 128), h0[d],
                         wroots, brels, wnext)
        h1[d] = res[0]
        for i, dd in enumerate(("H", "C")):
            g1[(d, dd)] = res[1 + i]

    rels1 = [(s, d) for (s, d) in edges if d != "Others"]
    g_cat1, src_cat1, dst_cat1, nck1, acc_total1, meta1 = _prep_layer(
        rels1, g1, edges, n_node, ("H", "C"))
    sc1 = _make_sc_scatter(nck1, acc_total1, meta1)
    P1 = sc1(g_cat1, src_cat1, dst_cat1)
    P1 = {d: P1[i] for i, d in enumerate(("H", "C"))}

    ys = {}
    for d in ("H", "C"):
        wroots = [_bd8(p["conv1_%s_%s_Wroot" % (s, d)]) for s in _TYPES]
        brels = [_t8(p["conv1_%s_%s_brel" % (s, d)]) for s in _TYPES]
        res = _comb_call(P1[d].reshape(2, -1, 128), h1[d],
                         wroots, brels, [],
                         Wh=_bd8(p["head_%s_W" % d]),
                         bh=_t8(p["head_%s_b" % d]))
        ys[d] = res[-1].reshape(-1, 1)
    return ys["H"], ys["C"]


# speedup vs baseline: 11.3105x; 1.0267x over previous
import functools

import jax
import jax.numpy as jnp
from jax import lax
from jax.experimental import pallas as pl
from jax.experimental.pallas import tpu as pltpu
from jax.experimental.pallas import tpu_sc as plsc

_TYPES = ("H", "C", "Others")
_F = 16
_NC = 2
_NS = 16
_NW = _NC * _NS
_CHUNK = 128
_DEPTH = 8
_SEG = 16
_EPB = _CHUNK * _NW * 4
_ZROWS = 128


def _acc_rows(n):
    return ((n + 8 + 127) // 128) * 128


def _make_sc_scatter(nck, acc_total, outs_meta):
    mesh = plsc.VectorSubcoreMesh(core_axis_name="c", subcore_axis_name="s")
    out_type = [jax.ShapeDtypeStruct((2, r, _F), jnp.float32)
                for (_, r) in outs_meta]
    scratch = [pltpu.VMEM((_ZROWS, _F), jnp.float32),
               pltpu.VMEM((2, _SEG * _CHUNK), jnp.int32),
               pltpu.VMEM((2, _SEG * _CHUNK), jnp.int32),
               pltpu.VMEM((_DEPTH, _CHUNK, _F), jnp.float32),
               pltpu.VMEM_SHARED((acc_total, _F), jnp.float32),
               pltpu.SemaphoreType.DMA, pltpu.SemaphoreType.DMA,
               pltpu.SemaphoreType.DMA, pltpu.SemaphoreType.DMA]
    D = len(outs_meta)

    def body(g, srcx, dstx, *refs):
        outs = refs[0:D]
        zbuf, sidx, didx, rows, acc, gsem, ssem, ib0, ib1 = refs[D:]
        isem = (ib0, ib1)

        cid = lax.axis_index("c")
        sid = lax.axis_index("s")
        wid = sid * _NC + cid
        m = nck * _CHUNK

        def _zb(i, c):
            zbuf[i, :] = jnp.zeros((_F,), jnp.float32)
            return c
        lax.fori_loop(0, _ZROWS, _zb, 0)
        per = acc_total // _NS
        for off in range(0, per, _ZROWS):
            c = min(_ZROWS, per - off)
            pltpu.sync_copy(zbuf.at[pl.ds(0, c)],
                            acc.at[pl.ds(sid * per + off, c)])
        plsc.subcore_barrier()

        _SEGC = _SEG * _CHUNK

        def _do_group(bi, j0, depth):
            gds = []
            for b in range(depth):
                o = (j0 + b) * _CHUNK
                gds.append(pltpu.async_copy(
                    g.at[sidx.at[bi, pl.ds(o, _CHUNK)]], rows.at[b], gsem))
            sds = []
            for b in range(depth):
                o = (j0 + b) * _CHUNK
                gds[b].wait()
                sds.append(pltpu.async_copy(
                    rows.at[b], acc.at[didx.at[bi, pl.ds(o, _CHUNK)]],
                    ssem, add=True))
            for b in range(depth):
                sds[b].wait()

        def _preload(bi, si, ne=_SEGC):
            e0 = wid * m + si * _SEGC
            pltpu.async_copy(srcx.at[pl.ds(e0, ne)],
                             sidx.at[bi, pl.ds(0, ne)], isem[bi])
            pltpu.async_copy(dstx.at[pl.ds(e0, ne)],
                             didx.at[bi, pl.ds(0, ne)], isem[bi])

        def _drain(bi, ne=_SEGC):
            pltpu.make_async_copy(srcx.at[pl.ds(0, ne)],
                                  sidx.at[bi, pl.ds(0, ne)],
                                  isem[bi]).wait()
            pltpu.make_async_copy(dstx.at[pl.ds(0, ne)],
                                  didx.at[bi, pl.ds(0, ne)],
                                  isem[bi]).wait()

        def _process(bi, sn):
            full = sn // _DEPTH
            tail = sn - full * _DEPTH
            if full:
                lax.fori_loop(
                    0, full,
                    lambda j, c: (_do_group(bi, j * _DEPTH, _DEPTH), c)[1],
                    0)
            if tail:
                _do_group(bi, full * _DEPTH, tail)

        n_full = nck // _SEG
        tail_ck = nck - n_full * _SEG
        npairs = n_full // 2
        clamp = n_full - 1

        if n_full:
            _preload(0, 0)
            _preload(1, jnp.minimum(1, clamp))

            def _pair(t, c):
                _drain(0)
                _process(0, _SEG)
                _preload(0, jnp.minimum(2 * t + 2, clamp))
                _drain(1)
                _process(1, _SEG)
                _preload(1, jnp.minimum(2 * t + 3, clamp))
                return c
            lax.fori_loop(0, npairs, _pair, 0)
            _drain(0)
            if n_full % 2:
                _process(0, _SEG)
            _drain(1)
        if tail_ck:
            ne = tail_ck * _CHUNK
            e0 = wid * m + n_full * _SEGC
            pltpu.async_copy(srcx.at[pl.ds(e0, ne)],
                             sidx.at[0, pl.ds(0, ne)], isem[0])
            pltpu.async_copy(dstx.at[pl.ds(e0, ne)],
                             didx.at[0, pl.ds(0, ne)], isem[0])
            _drain(0, ne)
            _process(0, tail_ck)
        plsc.subcore_barrier()

        for o_i, (a0, r) in enumerate(outs_meta):
            per_d = r // _NS
            for off in range(0, per_d, _ZROWS):
                c = min(_ZROWS, per_d - off)
                r0 = sid * per_d + off
                pltpu.sync_copy(acc.at[pl.ds(a0 + r0, c)],
                                zbuf.at[pl.ds(0, c)])
                pltpu.sync_copy(zbuf.at[pl.ds(0, c)],
                                outs[o_i].at[cid, pl.ds(r0, c)])

    return pl.kernel(body, out_type=out_type, mesh=mesh,
                     scratch_types=scratch,
                     compiler_params=pltpu.CompilerParams(
                         use_tc_tiling_on_sc=False))


_TPB = 256
_PK = 8


def _bd8(W):
    return jnp.kron(jnp.eye(_PK, dtype=W.dtype), W)


def _t8(b):
    return jnp.tile(b.reshape(-1), _PK).reshape(1, -1)


def _enc_call(xp, W1, b1, W2, b2, Wrels):
    M = xp.shape[0]
    k = len(Wrels)

    def body(x_ref, w1_ref, b1_ref, w2_ref, b2_ref, *rest):
        wr = rest[:k]
        h_ref = rest[k]
        gs = rest[k + 1:]
        z = jnp.maximum(
            jnp.dot(x_ref[...], w1_ref[...],
                    preferred_element_type=jnp.float32) + b1_ref[...], 0.0)
        h = jnp.maximum(
            jnp.dot(z, w2_ref[...],
                    preferred_element_type=jnp.float32) + b2_ref[...], 0.0)
        h_ref[...] = h
        for i in range(k):
            gs[i][...] = jnp.dot(h, wr[i][...],
                                 preferred_element_type=jnp.float32)

    in_specs = ([pl.BlockSpec((_TPB, 1024), lambda i: (i, 0)),
                 pl.BlockSpec((1024, 256), lambda i: (0, 0)),
                 pl.BlockSpec((1, 256), lambda i: (0, 0)),
                 pl.BlockSpec((256, 128), lambda i: (0, 0)),
                 pl.BlockSpec((1, 128), lambda i: (0, 0))]
                + [pl.BlockSpec((128, 128), lambda i: (0, 0))] * k)
    out_specs = [pl.BlockSpec((_TPB, 128), lambda i: (i, 0))] * (1 + k)
    out_shape = [jax.ShapeDtypeStruct((M, 128), jnp.float32)] * (1 + k)
    return pl.pallas_call(body, grid=(-(-M // _TPB),), in_specs=in_specs,
                          out_specs=out_specs, out_shape=out_shape)(
        xp, W1, b1, W2, b2, *Wrels)


def _comb_call(P, h, Wroots, brels, Wnext, Wh=None, bh=None):
    M = h.shape[0]
    nb = -(-M // _TPB)
    k = len(Wnext)
    head = Wh is not None

    def body(p0_ref, p1_ref, h_ref, wr0, wr1, wr2, c0, c1, c2, *rest):
        wn = rest[:k]
        rest = rest[k:]
        if head:
            wh_ref, bh_ref = rest[0], rest[1]
            rest = rest[2:]
        h_out = rest[0]
        gs = rest[1:1 + k]
        W = wr0[...] + wr1[...] + wr2[...]
        bb = c0[...] + c1[...] + c2[...]
        m = (p0_ref[0, :, :] + p1_ref[0, :, :]
             + jnp.dot(h_ref[...], W, preferred_element_type=jnp.float32)
             + bb)
        hn = jnp.maximum(m, 0.0)
        h_out[...] = hn
        for i in range(k):
            gs[i][...] = jnp.dot(hn, wn[i][...],
                                 preferred_element_type=jnp.float32)
        if head:
            y_ref = rest[1 + k]
            y_ref[...] = jnp.dot(hn, wh_ref[...],
                                 preferred_element_type=jnp.float32) \
                + bh_ref[...]

    in_specs = ([pl.BlockSpec((1, _TPB, 128), lambda i: (0, i, 0)),
                 pl.BlockSpec((1, _TPB, 128), lambda i: (1, i, 0)),
                 pl.BlockSpec((_TPB, 128), lambda i: (i, 0))]
                + [pl.BlockSpec((128, 128), lambda i: (0, 0))] * 3
                + [pl.BlockSpec((1, 128), lambda i: (0, 0))] * 3
                + [pl.BlockSpec((128, 128), lambda i: (0, 0))] * k)
    out_specs = [pl.BlockSpec((_TPB, 128), lambda i: (i, 0))] * (1 + k)
    out_shape = [jax.ShapeDtypeStruct((M, 128), jnp.float32)] * (1 + k)
    args = [P, P, h] + list(Wroots) + list(brels) + list(Wnext)
    if head:
        in_specs += [pl.BlockSpec((128, _PK), lambda i: (0, 0)),
                     pl.BlockSpec((1, _PK), lambda i: (0, 0))]
        out_specs = out_specs + [pl.BlockSpec((_TPB, _PK),
                                              lambda i: (i, 0))]
        out_shape = out_shape + [jax.ShapeDtypeStruct((M, _PK),
                                                      jnp.float32)]
        args += [Wh, bh]
    return pl.pallas_call(body, grid=(nb,), in_specs=in_specs,
                          out_specs=out_specs, out_shape=out_shape)(*args)


def _prep_layer(rel_list, g_dict, edges, n_node, dsts):
    acc_off = {}
    off = 0
    for d in dsts:
        acc_off[d] = off
        off += _acc_rows(n_node[d])
    acc_total = off
    g_parts, src_parts, dst_parts = [], [], []
    gbase = 0
    for (s, d) in rel_list:
        ei = edges[(s, d)]
        g_parts.append(g_dict[(s, d)])
        src_parts.append(ei[0].astype(jnp.int32) + gbase)
        dst_parts.append(ei[1].astype(jnp.int32) + acc_off[d])
        gbase += n_node[s]
    E = sum(edges[r].shape[1] for r in rel_list)
    ng = -(-E // _EPB)
    nck = ng * 4
    pad = ng * _EPB - E
    trash = acc_off[dsts[0]] + n_node[dsts[0]]
    src_cat = jnp.concatenate(src_parts + [jnp.zeros((pad,), jnp.int32)])
    dst_cat = jnp.concatenate(dst_parts
                              + [jnp.full((pad,), trash, jnp.int32)])
    g_cat = jnp.concatenate(g_parts, axis=0).reshape(-1, _F)
    outs_meta = [(acc_off[d], _acc_rows(n_node[d])) for d in dsts]
    return g_cat, src_cat, dst_cat, nck, acc_total, outs_meta


def kernel(x_H, x_C, x_Others,
           edge_index_H_H, edge_index_H_C, edge_index_H_Others,
           edge_index_C_H, edge_index_C_C, edge_index_C_Others,
           edge_index_Others_H, edge_index_Others_C,
           edge_index_Others_Others, params):
    p = params
    x = {"H": x_H, "C": x_C, "Others": x_Others}
    edges = {
        ("H", "H"): edge_index_H_H,
        ("H", "C"): edge_index_H_C,
        ("H", "Others"): edge_index_H_Others,
        ("C", "H"): edge_index_C_H,
        ("C", "C"): edge_index_C_C,
        ("C", "Others"): edge_index_C_Others,
        ("Others", "H"): edge_index_Others_H,
        ("Others", "C"): edge_index_Others_C,
        ("Others", "Others"): edge_index_Others_Others,
    }
    n_node = {t: x[t].shape[0] for t in _TYPES}

    h0 = {}
    g0 = {}
    for t in _TYPES:
        wr = [_bd8(p["conv0_%s_%s_Wrel" % (t, d)]) for d in _TYPES]
        res = _enc_call(x[t].reshape(-1, 128 * _PK),
                        _bd8(p["enc_%s_W1" % t]), _t8(p["enc_%s_b1" % t]),
                        _bd8(p["enc_%s_W2" % t]), _t8(p["enc_%s_b2" % t]),
                        wr)
        h0[t] = res[0]
        for i, d in enumerate(_TYPES):
            g0[(t, d)] = res[1 + i]

    rels0 = [(s, d) for (s, d) in edges]
    g_cat, src_cat, dst_cat, nck, acc_total, meta = _prep_layer(
        rels0, g0, edges, n_node, _TYPES)
    sc0 = _make_sc_scatter(nck, acc_total, meta)
    P0 = sc0(g_cat, src_cat, dst_cat)
    P0 = {d: P0[i] for i, d in enumerate(_TYPES)}

    h1 = {}
    g1 = {}
    for d in _TYPES:
        wroots = [_bd8(p["conv0_%s_%s_Wroot" % (s, d)]) for s in _TYPES]
        brels = [_t8(p["conv0_%s_%s_brel" % (s, d)]) for s in _TYPES]
        wnext = [_bd8(p["conv1_%s_%s_Wrel" % (d, dd)])
                 for dd in ("H", "C")]
        res = _comb_call(P0[d].reshape(2, -1, 128), h0[d],
                         wroots, brels, wnext)
        h1[d] = res[0]
        for i, dd in enumerate(("H", "C")):
            g1[(d, dd)] = res[1 + i]

    rels1 = [(s, d) for (s, d) in edges if d != "Others"]
    g_cat1, src_cat1, dst_cat1, nck1, acc_total1, meta1 = _prep_layer(
        rels1, g1, edges, n_node, ("H", "C"))
    sc1 = _make_sc_scatter(nck1, acc_total1, meta1)
    P1 = sc1(g_cat1, src_cat1, dst_cat1)
    P1 = {d: P1[i] for i, d in enumerate(("H", "C"))}

    ys = {}
    for d in ("H", "C"):
        wroots = [_bd8(p["conv1_%s_%s_Wroot" % (s, d)]) for s in _TYPES]
        brels = [_t8(p["conv1_%s_%s_brel" % (s, d)]) for s in _TYPES]
        res = _comb_call(P1[d].reshape(2, -1, 128), h1[d],
                         wroots, brels, [],
                         Wh=_bd8(p["head_%s_W" % d]),
                         bh=_t8(p["head_%s_b" % d]))
        ys[d] = res[-1].reshape(-1, 1)
    return ys["H"], ys["C"]
